# Initial kernel scaffold; baseline (speedup 1.0000x reference)
import jax
import jax.numpy as jnp
from jax.experimental import pallas as pl


def kernel(features, edge_index, W1, b1, W2, b2, W3, b3):
    raise NotImplementedError("write your pallas kernel here")


# speedup vs baseline: 5.4232x; 5.4232x over previous
import functools

import jax
import jax.numpy as jnp
from jax import lax
from jax.experimental import pallas as pl
from jax.experimental.pallas import tpu as pltpu
from jax.experimental.pallas import tpu_sc as plsc

N_NODES = 10000
N_EDGES = 320000
F_IN = 128
F_HID = 128
F_OUT = 40
F_OUT_PAD = 128

NC = 2
NS = 16
CHUNK = 128
N_CHUNKS = N_EDGES // CHUNK
CHUNKS_PER_SC = N_CHUNKS // NC
NODE_SLOTS = 5
N_PAD = NODE_SLOTS * NS * CHUNK

ROW_BLK = 1000


def _sc_mesh():
    return plsc.VectorSubcoreMesh(
        core_axis_name="c", subcore_axis_name="s", num_cores=NC, num_subcores=NS
    )


def _fill_zero_2d(buf, rows, d):
    z = jnp.zeros((16,), jnp.float32)

    def row(i, carry):
        for j in range(d // 16):
            buf[i, pl.ds(j * 16, 16)] = z
        return carry

    lax.fori_loop(0, rows, row, 0)


def _make_agg(d):
    @functools.partial(
        pl.kernel,
        out_type=jax.ShapeDtypeStruct((NC, N_PAD, d), jnp.float32),
        mesh=_sc_mesh(),
        scratch_types=[
            pltpu.VMEM((CHUNK,), jnp.int32),
            pltpu.VMEM((1, CHUNK), jnp.int32),
            pltpu.VMEM((CHUNK, d), jnp.float32),
            pltpu.VMEM_SHARED((N_PAD, d), jnp.float32),
        ],
    )
    def agg(table, edges, out, sidx, didx, gbuf, acc):
        c = lax.axis_index("c")
        s = lax.axis_index("s")

        _fill_zero_2d(gbuf, CHUNK, d)

        def zero_chunk(k, carry):
            t = s + NS * k
            pltpu.sync_copy(gbuf, acc.at[pl.ds(t * CHUNK, CHUNK)])
            return carry

        lax.fori_loop(0, NODE_SLOTS, zero_chunk, 0)
        plsc.subcore_barrier()

        nk = (CHUNKS_PER_SC - s + NS - 1) // NS

        def body(k, carry):
            e0 = (c * CHUNKS_PER_SC + s + k * NS) * CHUNK
            pltpu.sync_copy(edges.at[0, pl.ds(e0, CHUNK)], sidx)
            pltpu.sync_copy(edges.at[1, pl.ds(e0, CHUNK)], didx.at[0])
            pltpu.sync_copy(table.at[sidx], gbuf)
            pltpu.sync_copy(gbuf, acc.at[didx.at[0]], add=True)
            return carry

        lax.fori_loop(0, nk, body, 0)
        plsc.subcore_barrier()

        def wb_chunk(k, carry):
            r0 = (s + NS * k) * CHUNK
            pltpu.sync_copy(acc.at[pl.ds(r0, CHUNK)], gbuf)
            pltpu.sync_copy(gbuf, out.at[c, pl.ds(r0, CHUNK)])
            return carry

        lax.fori_loop(0, NODE_SLOTS, wb_chunk, 0)

    return agg


_agg_hid = _make_agg(F_HID)
_agg_out = _make_agg(F_OUT_PAD)


@functools.partial(
    pl.kernel,
    out_type=(
        jax.ShapeDtypeStruct((NC, N_PAD), jnp.float32),
        jax.ShapeDtypeStruct((NC, N_PAD), jnp.float32),
    ),
    mesh=_sc_mesh(),
    scratch_types=[
        pltpu.VMEM((1, CHUNK), jnp.int32),
        pltpu.VMEM((1, CHUNK), jnp.int32),
        pltpu.VMEM((CHUNK,), jnp.float32),
        pltpu.VMEM((CHUNK,), jnp.float32),
        pltpu.VMEM_SHARED((N_PAD,), jnp.float32),
        pltpu.VMEM_SHARED((N_PAD,), jnp.float32),
    ],
)
def _deg(edges, out_o, out_i, sidx, didx, ones, buf, acc_o, acc_i):
    c = lax.axis_index("c")
    s = lax.axis_index("s")
    one = jnp.ones((16,), jnp.float32)
    z = jnp.zeros((16,), jnp.float32)
    for j in range(CHUNK // 16):
        ones[pl.ds(j * 16, 16)] = one
        buf[pl.ds(j * 16, 16)] = z

    def zero_chunk(k, carry):
        t = s + NS * k
        pltpu.sync_copy(buf, acc_o.at[pl.ds(t * CHUNK, CHUNK)])
        pltpu.sync_copy(buf, acc_i.at[pl.ds(t * CHUNK, CHUNK)])
        return carry

    lax.fori_loop(0, NODE_SLOTS, zero_chunk, 0)
    plsc.subcore_barrier()

    nk = (CHUNKS_PER_SC - s + NS - 1) // NS

    def body(k, carry):
        e0 = (c * CHUNKS_PER_SC + s + k * NS) * CHUNK
        pltpu.sync_copy(edges.at[0, pl.ds(e0, CHUNK)], sidx.at[0])
        pltpu.sync_copy(edges.at[1, pl.ds(e0, CHUNK)], didx.at[0])
        pltpu.sync_copy(ones, acc_o.at[sidx.at[0]], add=True)
        pltpu.sync_copy(ones, acc_i.at[didx.at[0]], add=True)
        return carry

    lax.fori_loop(0, nk, body, 0)
    plsc.subcore_barrier()

    def wb_chunk(k, carry):
        r0 = (s + NS * k) * CHUNK
        pltpu.sync_copy(acc_o.at[pl.ds(r0, CHUNK)], buf)
        pltpu.sync_copy(buf, out_o.at[c, pl.ds(r0, CHUNK)])
        pltpu.sync_copy(acc_i.at[pl.ds(r0, CHUNK)], buf)
        pltpu.sync_copy(buf, out_i.at[c, pl.ds(r0, CHUNK)])
        return carry

    lax.fori_loop(0, NODE_SLOTS, wb_chunk, 0)


def _norm(deg):
    return jnp.where(deg > 0, lax.rsqrt(jnp.maximum(deg, 1.0)), 0.0)


def _tc_first(features, deg_out, W):
    def body(x_ref, d_ref, w_ref, o_ref):
        ns = _norm(d_ref[...])
        o_ref[...] = jnp.dot(
            x_ref[...] * ns, w_ref[...], preferred_element_type=jnp.float32
        )

    return pl.pallas_call(
        body,
        grid=(N_NODES // ROW_BLK,),
        in_specs=[
            pl.BlockSpec((ROW_BLK, F_IN), lambda i: (i, 0)),
            pl.BlockSpec((ROW_BLK, 1), lambda i: (i, 0)),
            pl.BlockSpec((F_IN, F_HID), lambda i: (0, 0)),
        ],
        out_specs=pl.BlockSpec((ROW_BLK, F_HID), lambda i: (i, 0)),
        out_shape=jax.ShapeDtypeStruct((N_NODES, F_HID), jnp.float32),
    )(features, deg_out, W)


def _tc_mid(parts, deg_in, b, deg_out, W, d_out):
    def body(p_ref, di_ref, b_ref, do_ref, w_ref, o_ref):
        nd = _norm(di_ref[...])
        ns = _norm(do_ref[...])
        h = (p_ref[0] + p_ref[1]) * nd + b_ref[...]
        h = jnp.maximum(h, 0.0)
        o_ref[...] = jnp.dot(h * ns, w_ref[...], preferred_element_type=jnp.float32)

    return pl.pallas_call(
        body,
        grid=(N_NODES // ROW_BLK,),
        in_specs=[
            pl.BlockSpec((NC, ROW_BLK, F_HID), lambda i: (0, i, 0)),
            pl.BlockSpec((ROW_BLK, 1), lambda i: (i, 0)),
            pl.BlockSpec((1, F_HID), lambda i: (0, 0)),
            pl.BlockSpec((ROW_BLK, 1), lambda i: (i, 0)),
            pl.BlockSpec((F_HID, d_out), lambda i: (0, 0)),
        ],
        out_specs=pl.BlockSpec((ROW_BLK, d_out), lambda i: (i, 0)),
        out_shape=jax.ShapeDtypeStruct((N_NODES, d_out), jnp.float32),
    )(parts, deg_in, b, deg_out, W)


def _tc_final(parts, deg_in, b):
    def body(p_ref, di_ref, b_ref, o_ref):
        nd = _norm(di_ref[...])
        o_ref[...] = (p_ref[0] + p_ref[1]) * nd + b_ref[...]

    return pl.pallas_call(
        body,
        grid=(N_NODES // ROW_BLK,),
        in_specs=[
            pl.BlockSpec((NC, ROW_BLK, F_OUT_PAD), lambda i: (0, i, 0)),
            pl.BlockSpec((ROW_BLK, 1), lambda i: (i, 0)),
            pl.BlockSpec((1, F_OUT_PAD), lambda i: (0, 0)),
        ],
        out_specs=pl.BlockSpec((ROW_BLK, F_OUT_PAD), lambda i: (i, 0)),
        out_shape=jax.ShapeDtypeStruct((N_NODES, F_OUT_PAD), jnp.float32),
    )(parts, deg_in, b)


def kernel(features, edge_index, W1, b1, W2, b2, W3, b3):
    W3p = jnp.pad(W3, ((0, 0), (0, F_OUT_PAD - F_OUT)))
    b3p = jnp.pad(b3, (0, F_OUT_PAD - F_OUT))

    do_parts, di_parts = _deg(edge_index)
    deg_out = (do_parts[0, :N_NODES] + do_parts[1, :N_NODES]).reshape(N_NODES, 1)
    deg_in = (di_parts[0, :N_NODES] + di_parts[1, :N_NODES]).reshape(N_NODES, 1)

    h = _tc_first(features, deg_out, W1)
    parts = _agg_hid(h, edge_index)[:, :N_NODES]
    h = _tc_mid(parts, deg_in, b1.reshape(1, -1), deg_out, W2, F_HID)
    parts = _agg_hid(h, edge_index)[:, :N_NODES]
    h = _tc_mid(parts, deg_in, b2.reshape(1, -1), deg_out, W3p, F_OUT_PAD)
    parts = _agg_out(h, edge_index)[:, :N_NODES]
    out = _tc_final(parts, deg_in, b3p.reshape(1, -1))
    return out[:, :F_OUT]
